# baseline (speedup 1.0000x reference)
import functools

import jax
import jax.numpy as jnp
from jax import lax
from jax.experimental import pallas as pl
from jax.experimental.pallas import tpu as pltpu
from jax.experimental.pallas import tpu_sc as plsc

N = 10000
NP = 10240
E = 100000
EP = 106496
NC, NS = 2, 16
NW = NC * NS
IRPW = 26
RPT = NP // NS

IN, HID, OUT = 128, 128, 64

_MESH = plsc.VectorSubcoreMesh(
    core_axis_name="c", subcore_axis_name="s", num_cores=NC, num_subcores=NS)


EPW = EP // NW


def _deg_body(s0, d0, s1, d1, s2, d2, out, idx_v, hist_v):
    cid = lax.axis_index("c")
    sid = lax.axis_index("s")
    w = cid * NS + sid
    zeros16 = jnp.zeros((16,), jnp.float32)
    ones16 = jnp.ones((16,), jnp.float32)

    def zbody(i, _):
        hist_v[pl.ds(i * 16, 16)] = zeros16
        return 0
    lax.fori_loop(0, 6 * NP // 16, zbody, 0)

    for h, arr in enumerate((s0, d0, s1, d1, s2, d2)):
        pltpu.sync_copy(arr.at[pl.ds(w * EPW, EPW)], idx_v)

        def gbody(g, _, h=h):
            vec = idx_v[pl.ds(g * 16, 16)]
            plsc.addupdate_scatter(hist_v, [vec + jnp.int32(h * NP)], ones16)
            return 0
        lax.fori_loop(0, EPW // 16, gbody, 0)

    pltpu.sync_copy(hist_v, out.at[pl.ds(w * 6 * NP, 6 * NP)])


_sc_degrees = pl.kernel(
    _deg_body,
    out_type=jax.ShapeDtypeStruct((NW * 6 * NP,), jnp.float32),
    mesh=_MESH,
    compiler_params=pltpu.CompilerParams(needs_layout_passes=False),
    scratch_types=[
        pltpu.VMEM((EPW,), jnp.int32),
        pltpu.VMEM((6 * NP,), jnp.float32),
    ],
)


def _scat_body(s0, d0, s1, d1, s2, d2, hn0, hn1, hn2,
                 out_a, out_b, idxs_v, idxd_v, rows_v, zeros_v, acc_sh,
                 sem, gsem0, gsem1, ssem0, ssem1):
    cid = lax.axis_index("c")
    sid = lax.axis_index("s")
    w = cid * NS + sid
    zeros16 = jnp.zeros((16,), jnp.float32)
    gsem = (gsem0, gsem1)
    ssem = (ssem0, ssem1)

    def zbody(i, _):
        zeros_v[i // 8, pl.ds((i % 8) * 16, 16)] = zeros16
        return 0
    lax.fori_loop(0, 32 * 8, zbody, 0)

    zds = [pltpu.async_copy(
               zeros_v, acc_sh.at[pl.ds(sid * RPT + j * 32, 32)], sem)
           for j in range(RPT // 32)]
    for d in zds:
        d.wait()

    for r, (se, de, hn) in enumerate(((s0, d0, hn0), (s1, d1, hn1),
                                      (s2, d2, hn2))):
        pltpu.sync_copy(se.at[w], idxs_v)
        pltpu.sync_copy(de.at[w], idxd_v)

        def fire_g(i, b, hn=hn):
            pltpu.async_copy(hn.at[idxs_v.at[i]],
                             rows_v.at[pl.ds(b * 128, 128)], gsem[b])

        def fire_s(i, b):
            pltpu.async_copy(rows_v.at[pl.ds(b * 128, 128)],
                             acc_sh.at[idxd_v.at[i]], ssem[b], add=True)

        def wait_g(b, hn=hn):
            pltpu.make_async_copy(hn.at[pl.ds(0, 128)],
                                  rows_v.at[pl.ds(b * 128, 128)],
                                  gsem[b]).wait()

        def wait_s(b):
            pltpu.make_async_copy(rows_v.at[pl.ds(b * 128, 128)],
                                  acc_sh.at[pl.ds(0, 128)], ssem[b]).wait()

        fire_g(0, 0)
        fire_g(1, 1)
        plsc.subcore_barrier()
        wait_g(0)
        fire_s(0, 0)

        def ebody(t, _):
            g1 = 2 * t + 1
            wait_g(1)
            wait_s(0)
            fire_g(g1 + 1, 0)
            fire_s(g1, 1)
            wait_g(0)
            wait_s(1)
            fire_g(g1 + 2, 1)
            fire_s(g1 + 1, 0)
            return 0
        lax.fori_loop(0, (IRPW - 2) // 2, ebody, 0)

        wait_g(1)
        wait_s(0)
        fire_s(IRPW - 1, 1)
        wait_s(1)
        plsc.subcore_barrier()

        @pl.when(cid == 0)
        def _():
            pltpu.sync_copy(acc_sh.at[pl.ds(sid * RPT, RPT)],
                            out_a.at[r, pl.ds(sid * RPT, RPT)])

        @pl.when(cid == 1)
        def _():
            pltpu.sync_copy(acc_sh.at[pl.ds(sid * RPT, RPT)],
                            out_b.at[r, pl.ds(sid * RPT, RPT)])

        if r < 2:
            zds = [pltpu.async_copy(
                       zeros_v, acc_sh.at[pl.ds(sid * RPT + j * 32, 32)], sem)
                   for j in range(RPT // 32)]
            for d in zds:
                d.wait()
        plsc.subcore_barrier()


def _make_scatter(width):
    return pl.kernel(
        _scat_body,
        out_type=[jax.ShapeDtypeStruct((3, NP, width), jnp.float32),
                  jax.ShapeDtypeStruct((3, NP, width), jnp.float32)],
        mesh=_MESH,
        compiler_params=pltpu.CompilerParams(
            use_tc_tiling_on_sc=(width == 128)),
        scratch_types=[
            pltpu.VMEM((IRPW, 128), jnp.int32),
            pltpu.VMEM((IRPW, 128), jnp.int32),
            pltpu.VMEM((256, width), jnp.float32),
            pltpu.VMEM((32, width), jnp.float32),
            pltpu.VMEM_SHARED((NP, width), jnp.float32),
            pltpu.SemaphoreType.DMA,
            pltpu.SemaphoreType.DMA,
            pltpu.SemaphoreType.DMA,
            pltpu.SemaphoreType.DMA,
            pltpu.SemaphoreType.DMA,
        ],
    )


_sc_scatter = _make_scatter(HID)
_sc_scatter64 = _make_scatter(OUT)


_BLK = 512
_NBLK = NP // _BLK


def _prep_body(parts_ref, x_ref, cm_ref, hn0_ref, hn1_ref, hn2_ref):
    deg = jnp.sum(parts_ref[...], axis=0)
    c = jnp.where(deg > 0, lax.rsqrt(deg), 0.0)
    c8 = jnp.concatenate(
        [c[1:2], c[3:4], c[5:6], c[0:1], c[2:3], c[4:5],
         jnp.zeros((2, NP), jnp.float32)], axis=0)
    cm = c8.T
    cm_ref[...] = cm
    x = x_ref[...]
    hn0_ref[...] = x * cm[:, 3:4]
    hn1_ref[...] = x * cm[:, 4:5]
    hn2_ref[...] = x * cm[:, 5:6]


_tc_prep = pl.pallas_call(
    _prep_body,
    out_shape=[jax.ShapeDtypeStruct((NP, 8), jnp.float32)]
    + [jax.ShapeDtypeStruct((NP, IN), jnp.float32)] * 3,
)


def _layer_compute(pa_ref, pb_ref, cm_ref, w_ref, b_ref):
    cm = cm_ref[...]
    p = pa_ref[...] + pb_ref[...]
    w = w_ref[...]
    h = (jnp.dot(p[0] * cm[:, 0:1], w[0], preferred_element_type=jnp.float32)
         + jnp.dot(p[1] * cm[:, 1:2], w[1], preferred_element_type=jnp.float32)
         + jnp.dot(p[2] * cm[:, 2:3], w[2], preferred_element_type=jnp.float32)
         + jnp.sum(b_ref[...], axis=0, keepdims=True))
    return h, cm


def _layer_body(pa_ref, pb_ref, cm_ref, w_ref, b_ref,
                hn0_ref, hn1_ref, hn2_ref):
    h, cm = _layer_compute(pa_ref, pb_ref, cm_ref, w_ref, b_ref)
    h = jnp.maximum(h, 0.0)
    hn0_ref[...] = h * cm[:, 3:4]
    hn1_ref[...] = h * cm[:, 4:5]
    hn2_ref[...] = h * cm[:, 5:6]


_tc_layer = pl.pallas_call(
    _layer_body,
    grid=(_NBLK,),
    in_specs=[pl.BlockSpec((3, _BLK, HID), lambda i: (0, i, 0)),
              pl.BlockSpec((3, _BLK, HID), lambda i: (0, i, 0)),
              pl.BlockSpec((_BLK, 8), lambda i: (i, 0)),
              pl.BlockSpec((3, HID, HID), lambda i: (0, 0, 0)),
              pl.BlockSpec((3, HID), lambda i: (0, 0))],
    out_specs=[pl.BlockSpec((_BLK, HID), lambda i: (i, 0))] * 3,
    out_shape=[jax.ShapeDtypeStruct((NP, HID), jnp.float32)] * 3,
)


def _layer2m_body(pa_ref, pb_ref, cm_ref, w_ref, b_ref, w3_ref,
                  m0_ref, m1_ref, m2_ref):
    h, cm = _layer_compute(pa_ref, pb_ref, cm_ref, w_ref, b_ref)
    h = jnp.maximum(h, 0.0)
    w3 = w3_ref[...]
    m0_ref[...] = jnp.dot(h * cm[:, 3:4], w3[0],
                          preferred_element_type=jnp.float32)
    m1_ref[...] = jnp.dot(h * cm[:, 4:5], w3[1],
                          preferred_element_type=jnp.float32)
    m2_ref[...] = jnp.dot(h * cm[:, 5:6], w3[2],
                          preferred_element_type=jnp.float32)


_tc_layer2m = pl.pallas_call(
    _layer2m_body,
    grid=(_NBLK,),
    in_specs=[pl.BlockSpec((3, _BLK, HID), lambda i: (0, i, 0)),
              pl.BlockSpec((3, _BLK, HID), lambda i: (0, i, 0)),
              pl.BlockSpec((_BLK, 8), lambda i: (i, 0)),
              pl.BlockSpec((3, HID, HID), lambda i: (0, 0, 0)),
              pl.BlockSpec((3, HID), lambda i: (0, 0)),
              pl.BlockSpec((3, HID, OUT), lambda i: (0, 0, 0))],
    out_specs=[pl.BlockSpec((_BLK, OUT), lambda i: (i, 0))] * 3,
    out_shape=[jax.ShapeDtypeStruct((NP, OUT), jnp.float32)] * 3,
)


def _fin_body(pa_ref, pb_ref, cm_ref, b_ref, h_ref):
    cm = cm_ref[...]
    p = pa_ref[...] + pb_ref[...]
    h_ref[...] = (p[0] * cm[:, 0:1] + p[1] * cm[:, 1:2] + p[2] * cm[:, 2:3]
                  + jnp.sum(b_ref[...], axis=0, keepdims=True))


_tc_fin = pl.pallas_call(
    _fin_body,
    grid=(_NBLK,),
    in_specs=[pl.BlockSpec((3, _BLK, OUT), lambda i: (0, i, 0)),
              pl.BlockSpec((3, _BLK, OUT), lambda i: (0, i, 0)),
              pl.BlockSpec((_BLK, 8), lambda i: (i, 0)),
              pl.BlockSpec((3, OUT), lambda i: (0, 0))],
    out_specs=pl.BlockSpec((_BLK, OUT), lambda i: (i, 0)),
    out_shape=jax.ShapeDtypeStruct((NP, OUT), jnp.float32),
)


def kernel(x, edge_index_r0, edge_index_r1, edge_index_r2,
           W0, b0, W1, b1, W2, b2, W3, b3):
    edges = []
    edges_flat = []
    pad_ids = (N + jnp.arange(EP - E, dtype=jnp.int32) % (NP - N))
    for ei in (edge_index_r0, edge_index_r1, edge_index_r2):
        for side in range(2):
            idx = jnp.concatenate([ei[side].astype(jnp.int32), pad_ids])
            edges_flat.append(idx)
            edges.append(idx.reshape(NW, IRPW, 128))

    parts = _sc_degrees(*edges_flat)
    xp = jnp.pad(x, ((0, NP - N), (0, 0)))
    cmat, hn0, hn1, hn2 = _tc_prep(parts.reshape(NW, 6, NP), xp)

    for l in range(2):
        pa, pb = _sc_scatter(*edges, hn0, hn1, hn2)
        hn0, hn1, hn2 = _tc_layer(pa, pb, cmat, (W0, W1)[l], (b0, b1)[l])
    pa, pb = _sc_scatter(*edges, hn0, hn1, hn2)
    m0, m1, m2 = _tc_layer2m(pa, pb, cmat, W2, b2, W3)
    pa, pb = _sc_scatter64(*edges, m0, m1, m2)
    h = _tc_fin(pa, pb, cmat, b3)
    return h[:N]


# speedup vs baseline: 1.0387x; 1.0387x over previous
import functools

import jax
import jax.numpy as jnp
from jax import lax
from jax.experimental import pallas as pl
from jax.experimental.pallas import tpu as pltpu
from jax.experimental.pallas import tpu_sc as plsc

N = 10000
NP = 10240
E = 100000
EP = 106496
NC, NS = 2, 16
NW = NC * NS
IRPW = 26
RPT = NP // NS

IN, HID, OUT = 128, 128, 64

_MESH = plsc.VectorSubcoreMesh(
    core_axis_name="c", subcore_axis_name="s", num_cores=NC, num_subcores=NS)


EPW = EP // NW


def _deg_body(s0, d0, s1, d1, s2, d2, out, idx_v, hist_v):
    cid = lax.axis_index("c")
    sid = lax.axis_index("s")
    w = cid * NS + sid
    zeros16 = jnp.zeros((16,), jnp.float32)
    ones16 = jnp.ones((16,), jnp.float32)

    def zbody(i, _):
        hist_v[pl.ds(i * 16, 16)] = zeros16
        return 0
    lax.fori_loop(0, 6 * NP // 16, zbody, 0)

    for h, arr in enumerate((s0, d0, s1, d1, s2, d2)):
        pltpu.sync_copy(arr.at[pl.ds(w * EPW, EPW)], idx_v)

        def gbody(g, _, h=h):
            vec = idx_v[pl.ds(g * 16, 16)]
            plsc.addupdate_scatter(hist_v, [vec + jnp.int32(h * NP)], ones16)
            return 0
        lax.fori_loop(0, EPW // 16, gbody, 0)

    pltpu.sync_copy(hist_v, out.at[pl.ds(w * 6 * NP, 6 * NP)])


_sc_degrees = pl.kernel(
    _deg_body,
    out_type=jax.ShapeDtypeStruct((NW * 6 * NP,), jnp.float32),
    mesh=_MESH,
    compiler_params=pltpu.CompilerParams(needs_layout_passes=False),
    scratch_types=[
        pltpu.VMEM((EPW,), jnp.int32),
        pltpu.VMEM((6 * NP,), jnp.float32),
    ],
)


def _scat_body(s0, d0, s1, d1, s2, d2, hn0, hn1, hn2,
                 out_a, out_b, idxs_v, idxd_v, rows_v, zeros_v, acc_sh,
                 sem, gsem0, gsem1, ssem0, ssem1):
    cid = lax.axis_index("c")
    sid = lax.axis_index("s")
    w = cid * NS + sid
    zeros16 = jnp.zeros((16,), jnp.float32)
    gsem = (gsem0, gsem1)
    ssem = (ssem0, ssem1)

    def zbody(i, _):
        zeros_v[i // 8, pl.ds((i % 8) * 16, 16)] = zeros16
        return 0
    lax.fori_loop(0, 32 * 8, zbody, 0)

    for r, (se, de, hn) in enumerate(((s0, d0, hn0), (s1, d1, hn1),
                                      (s2, d2, hn2))):
        zds = [pltpu.async_copy(
                   zeros_v, acc_sh.at[pl.ds(sid * RPT + j * 32, 32)], sem)
               for j in range(RPT // 32)]
        pltpu.sync_copy(se.at[w], idxs_v)
        pltpu.sync_copy(de.at[w], idxd_v)

        def fire_g(i, b, hn=hn):
            pltpu.async_copy(hn.at[idxs_v.at[i]],
                             rows_v.at[pl.ds(b * 128, 128)], gsem[b])

        def fire_s(i, b):
            pltpu.async_copy(rows_v.at[pl.ds(b * 128, 128)],
                             acc_sh.at[idxd_v.at[i]], ssem[b], add=True)

        def wait_g(b, hn=hn):
            pltpu.make_async_copy(hn.at[pl.ds(0, 128)],
                                  rows_v.at[pl.ds(b * 128, 128)],
                                  gsem[b]).wait()

        def wait_s(b):
            pltpu.make_async_copy(rows_v.at[pl.ds(b * 128, 128)],
                                  acc_sh.at[pl.ds(0, 128)], ssem[b]).wait()

        fire_g(0, 0)
        fire_g(1, 1)
        for d in zds:
            d.wait()
        plsc.subcore_barrier()
        wait_g(0)
        fire_s(0, 0)

        def ebody(t, _):
            g1 = 2 * t + 1
            wait_g(1)
            wait_s(0)
            fire_g(g1 + 1, 0)
            fire_s(g1, 1)
            wait_g(0)
            wait_s(1)
            fire_g(g1 + 2, 1)
            fire_s(g1 + 1, 0)
            return 0
        lax.fori_loop(0, (IRPW - 2) // 2, ebody, 0)

        wait_g(1)
        wait_s(0)
        fire_s(IRPW - 1, 1)
        wait_s(1)
        plsc.subcore_barrier()

        @pl.when(cid == 0)
        def _():
            pltpu.sync_copy(acc_sh.at[pl.ds(sid * RPT, RPT)],
                            out_a.at[r, pl.ds(sid * RPT, RPT)])

        @pl.when(cid == 1)
        def _():
            pltpu.sync_copy(acc_sh.at[pl.ds(sid * RPT, RPT)],
                            out_b.at[r, pl.ds(sid * RPT, RPT)])
        plsc.subcore_barrier()


def _make_scatter(width):
    return pl.kernel(
        _scat_body,
        out_type=[jax.ShapeDtypeStruct((3, NP, width), jnp.float32),
                  jax.ShapeDtypeStruct((3, NP, width), jnp.float32)],
        mesh=_MESH,
        compiler_params=pltpu.CompilerParams(
            use_tc_tiling_on_sc=(width == 128)),
        scratch_types=[
            pltpu.VMEM((IRPW, 128), jnp.int32),
            pltpu.VMEM((IRPW, 128), jnp.int32),
            pltpu.VMEM((256, width), jnp.float32),
            pltpu.VMEM((32, width), jnp.float32),
            pltpu.VMEM_SHARED((NP, width), jnp.float32),
            pltpu.SemaphoreType.DMA,
            pltpu.SemaphoreType.DMA,
            pltpu.SemaphoreType.DMA,
            pltpu.SemaphoreType.DMA,
            pltpu.SemaphoreType.DMA,
        ],
    )


_sc_scatter = _make_scatter(HID)
_sc_scatter64 = _make_scatter(OUT)


_BLK = 512
_NBLK = NP // _BLK


def _prep_body(parts_ref, x_ref, cm_ref, hn0_ref, hn1_ref, hn2_ref):
    deg = jnp.sum(parts_ref[...], axis=0)
    c = jnp.where(deg > 0, lax.rsqrt(deg), 0.0)
    c8 = jnp.concatenate(
        [c[1:2], c[3:4], c[5:6], c[0:1], c[2:3], c[4:5],
         jnp.zeros((2, NP), jnp.float32)], axis=0)
    cm = c8.T
    cm_ref[...] = cm
    x = x_ref[...]
    hn0_ref[...] = x * cm[:, 3:4]
    hn1_ref[...] = x * cm[:, 4:5]
    hn2_ref[...] = x * cm[:, 5:6]


_tc_prep = pl.pallas_call(
    _prep_body,
    out_shape=[jax.ShapeDtypeStruct((NP, 8), jnp.float32)]
    + [jax.ShapeDtypeStruct((NP, IN), jnp.float32)] * 3,
)


def _layer_compute(pa_ref, pb_ref, cm_ref, w_ref, b_ref):
    cm = cm_ref[...]
    p = pa_ref[...] + pb_ref[...]
    w = w_ref[...]
    h = (jnp.dot(p[0] * cm[:, 0:1], w[0], preferred_element_type=jnp.float32)
         + jnp.dot(p[1] * cm[:, 1:2], w[1], preferred_element_type=jnp.float32)
         + jnp.dot(p[2] * cm[:, 2:3], w[2], preferred_element_type=jnp.float32)
         + jnp.sum(b_ref[...], axis=0, keepdims=True))
    return h, cm


def _layer_body(pa_ref, pb_ref, cm_ref, w_ref, b_ref,
                hn0_ref, hn1_ref, hn2_ref):
    h, cm = _layer_compute(pa_ref, pb_ref, cm_ref, w_ref, b_ref)
    h = jnp.maximum(h, 0.0)
    hn0_ref[...] = h * cm[:, 3:4]
    hn1_ref[...] = h * cm[:, 4:5]
    hn2_ref[...] = h * cm[:, 5:6]


_tc_layer = pl.pallas_call(
    _layer_body,
    grid=(_NBLK,),
    in_specs=[pl.BlockSpec((3, _BLK, HID), lambda i: (0, i, 0)),
              pl.BlockSpec((3, _BLK, HID), lambda i: (0, i, 0)),
              pl.BlockSpec((_BLK, 8), lambda i: (i, 0)),
              pl.BlockSpec((3, HID, HID), lambda i: (0, 0, 0)),
              pl.BlockSpec((3, HID), lambda i: (0, 0))],
    out_specs=[pl.BlockSpec((_BLK, HID), lambda i: (i, 0))] * 3,
    out_shape=[jax.ShapeDtypeStruct((NP, HID), jnp.float32)] * 3,
)


def _layer2m_body(pa_ref, pb_ref, cm_ref, w_ref, b_ref, w3_ref,
                  m0_ref, m1_ref, m2_ref):
    h, cm = _layer_compute(pa_ref, pb_ref, cm_ref, w_ref, b_ref)
    h = jnp.maximum(h, 0.0)
    w3 = w3_ref[...]
    m0_ref[...] = jnp.dot(h * cm[:, 3:4], w3[0],
                          preferred_element_type=jnp.float32)
    m1_ref[...] = jnp.dot(h * cm[:, 4:5], w3[1],
                          preferred_element_type=jnp.float32)
    m2_ref[...] = jnp.dot(h * cm[:, 5:6], w3[2],
                          preferred_element_type=jnp.float32)


_tc_layer2m = pl.pallas_call(
    _layer2m_body,
    grid=(_NBLK,),
    in_specs=[pl.BlockSpec((3, _BLK, HID), lambda i: (0, i, 0)),
              pl.BlockSpec((3, _BLK, HID), lambda i: (0, i, 0)),
              pl.BlockSpec((_BLK, 8), lambda i: (i, 0)),
              pl.BlockSpec((3, HID, HID), lambda i: (0, 0, 0)),
              pl.BlockSpec((3, HID), lambda i: (0, 0)),
              pl.BlockSpec((3, HID, OUT), lambda i: (0, 0, 0))],
    out_specs=[pl.BlockSpec((_BLK, OUT), lambda i: (i, 0))] * 3,
    out_shape=[jax.ShapeDtypeStruct((NP, OUT), jnp.float32)] * 3,
)


def _fin_body(pa_ref, pb_ref, cm_ref, b_ref, h_ref):
    cm = cm_ref[...]
    p = pa_ref[...] + pb_ref[...]
    h_ref[...] = (p[0] * cm[:, 0:1] + p[1] * cm[:, 1:2] + p[2] * cm[:, 2:3]
                  + jnp.sum(b_ref[...], axis=0, keepdims=True))


_tc_fin = pl.pallas_call(
    _fin_body,
    grid=(_NBLK,),
    in_specs=[pl.BlockSpec((3, _BLK, OUT), lambda i: (0, i, 0)),
              pl.BlockSpec((3, _BLK, OUT), lambda i: (0, i, 0)),
              pl.BlockSpec((_BLK, 8), lambda i: (i, 0)),
              pl.BlockSpec((3, OUT), lambda i: (0, 0))],
    out_specs=pl.BlockSpec((_BLK, OUT), lambda i: (i, 0)),
    out_shape=jax.ShapeDtypeStruct((NP, OUT), jnp.float32),
)


def kernel(x, edge_index_r0, edge_index_r1, edge_index_r2,
           W0, b0, W1, b1, W2, b2, W3, b3):
    edges = []
    edges_flat = []
    pad_ids = (N + jnp.arange(EP - E, dtype=jnp.int32) % (NP - N))
    for ei in (edge_index_r0, edge_index_r1, edge_index_r2):
        for side in range(2):
            idx = jnp.concatenate([ei[side].astype(jnp.int32), pad_ids])
            edges_flat.append(idx)
            edges.append(idx.reshape(NW, IRPW, 128))

    parts = _sc_degrees(*edges_flat)
    xp = jnp.pad(x, ((0, NP - N), (0, 0)))
    cmat, hn0, hn1, hn2 = _tc_prep(parts.reshape(NW, 6, NP), xp)

    for l in range(2):
        pa, pb = _sc_scatter(*edges, hn0, hn1, hn2)
        hn0, hn1, hn2 = _tc_layer(pa, pb, cmat, (W0, W1)[l], (b0, b1)[l])
    pa, pb = _sc_scatter(*edges, hn0, hn1, hn2)
    m0, m1, m2 = _tc_layer2m(pa, pb, cmat, W2, b2, W3)
    pa, pb = _sc_scatter64(*edges, m0, m1, m2)
    h = _tc_fin(pa, pb, cmat, b3)
    return h[:N]


# speedup vs baseline: 1.0461x; 1.0071x over previous
import functools

import jax
import jax.numpy as jnp
from jax import lax
from jax.experimental import pallas as pl
from jax.experimental.pallas import tpu as pltpu
from jax.experimental.pallas import tpu_sc as plsc

N = 10000
NP = 10240
E = 100000
EP = 106496
NC, NS = 2, 16
NW = NC * NS
IRPW = 26
RPT = NP // NS

IN, HID, OUT = 128, 128, 64

_MESH = plsc.VectorSubcoreMesh(
    core_axis_name="c", subcore_axis_name="s", num_cores=NC, num_subcores=NS)


EPW = EP // NW


def _deg_body(s0, d0, s1, d1, s2, d2, out, idx0_v, idx1_v, hist_v, dsem):
    cid = lax.axis_index("c")
    sid = lax.axis_index("s")
    w = cid * NS + sid
    zeros16 = jnp.zeros((16,), jnp.float32)
    ones16 = jnp.ones((16,), jnp.float32)

    def zbody(i, _):
        hist_v[pl.ds(i * 16, 16)] = zeros16
        return 0
    lax.fori_loop(0, 6 * NP // 16, zbody, 0)

    arrs = (s0, d0, s1, d1, s2, d2)
    bufs = (idx0_v, idx1_v)
    pltpu.async_copy(arrs[0].at[pl.ds(w * EPW, EPW)], bufs[0], dsem)
    for h, arr in enumerate(arrs):
        buf = bufs[h % 2]
        pltpu.make_async_copy(arr.at[pl.ds(0, EPW)], buf, dsem).wait()
        if h + 1 < 6:
            pltpu.async_copy(arrs[h + 1].at[pl.ds(w * EPW, EPW)],
                             bufs[(h + 1) % 2], dsem)

        def gbody(g, _, h=h, buf=buf):
            vec0 = buf[pl.ds(g * 32, 16)]
            vec1 = buf[pl.ds(g * 32 + 16, 16)]
            plsc.addupdate_scatter(hist_v, [vec0 + jnp.int32(h * NP)], ones16)
            plsc.addupdate_scatter(hist_v, [vec1 + jnp.int32(h * NP)], ones16)
            return 0
        lax.fori_loop(0, EPW // 32, gbody, 0)

    pltpu.sync_copy(hist_v, out.at[pl.ds(w * 6 * NP, 6 * NP)])


_sc_degrees = pl.kernel(
    _deg_body,
    out_type=jax.ShapeDtypeStruct((NW * 6 * NP,), jnp.float32),
    mesh=_MESH,
    compiler_params=pltpu.CompilerParams(needs_layout_passes=False),
    scratch_types=[
        pltpu.VMEM((EPW,), jnp.int32),
        pltpu.VMEM((EPW,), jnp.int32),
        pltpu.VMEM((6 * NP,), jnp.float32),
        pltpu.SemaphoreType.DMA,
    ],
)


def _scat_body(s0, d0, s1, d1, s2, d2, hn0, hn1, hn2,
                 out_a, out_b, idxs_v, idxd_v, rows_v, zeros_v, acc_sh,
                 sem, gsem0, gsem1, ssem0, ssem1):
    cid = lax.axis_index("c")
    sid = lax.axis_index("s")
    w = cid * NS + sid
    zeros16 = jnp.zeros((16,), jnp.float32)
    gsem = (gsem0, gsem1)
    ssem = (ssem0, ssem1)

    def zbody(i, _):
        zeros_v[i // 8, pl.ds((i % 8) * 16, 16)] = zeros16
        return 0
    lax.fori_loop(0, 32 * 8, zbody, 0)

    for r, (se, de, hn) in enumerate(((s0, d0, hn0), (s1, d1, hn1),
                                      (s2, d2, hn2))):
        zds = [pltpu.async_copy(
                   zeros_v, acc_sh.at[pl.ds(sid * RPT + j * 32, 32)], sem)
               for j in range(RPT // 32)]
        pltpu.sync_copy(se.at[w], idxs_v)
        pltpu.sync_copy(de.at[w], idxd_v)

        def fire_g(i, b, hn=hn):
            pltpu.async_copy(hn.at[idxs_v.at[i]],
                             rows_v.at[pl.ds(b * 128, 128)], gsem[b])

        def fire_s(i, b):
            pltpu.async_copy(rows_v.at[pl.ds(b * 128, 128)],
                             acc_sh.at[idxd_v.at[i]], ssem[b], add=True)

        def wait_g(b, hn=hn):
            pltpu.make_async_copy(hn.at[pl.ds(0, 128)],
                                  rows_v.at[pl.ds(b * 128, 128)],
                                  gsem[b]).wait()

        def wait_s(b):
            pltpu.make_async_copy(rows_v.at[pl.ds(b * 128, 128)],
                                  acc_sh.at[pl.ds(0, 128)], ssem[b]).wait()

        fire_g(0, 0)
        fire_g(1, 1)
        for d in zds:
            d.wait()
        plsc.subcore_barrier()
        wait_g(0)
        fire_s(0, 0)

        def ebody(t, _):
            g1 = 2 * t + 1
            wait_g(1)
            wait_s(0)
            fire_g(g1 + 1, 0)
            fire_s(g1, 1)
            wait_g(0)
            wait_s(1)
            fire_g(g1 + 2, 1)
            fire_s(g1 + 1, 0)
            return 0
        lax.fori_loop(0, (IRPW - 2) // 2, ebody, 0)

        wait_g(1)
        wait_s(0)
        fire_s(IRPW - 1, 1)
        wait_s(1)
        plsc.subcore_barrier()

        @pl.when(cid == 0)
        def _():
            pltpu.sync_copy(acc_sh.at[pl.ds(sid * RPT, RPT)],
                            out_a.at[r, pl.ds(sid * RPT, RPT)])

        @pl.when(cid == 1)
        def _():
            pltpu.sync_copy(acc_sh.at[pl.ds(sid * RPT, RPT)],
                            out_b.at[r, pl.ds(sid * RPT, RPT)])
        plsc.subcore_barrier()


def _make_scatter(width):
    return pl.kernel(
        _scat_body,
        out_type=[jax.ShapeDtypeStruct((3, NP, width), jnp.float32),
                  jax.ShapeDtypeStruct((3, NP, width), jnp.float32)],
        mesh=_MESH,
        compiler_params=pltpu.CompilerParams(
            use_tc_tiling_on_sc=(width == 128)),
        scratch_types=[
            pltpu.VMEM((IRPW, 128), jnp.int32),
            pltpu.VMEM((IRPW, 128), jnp.int32),
            pltpu.VMEM((256, width), jnp.float32),
            pltpu.VMEM((32, width), jnp.float32),
            pltpu.VMEM_SHARED((NP, width), jnp.float32),
            pltpu.SemaphoreType.DMA,
            pltpu.SemaphoreType.DMA,
            pltpu.SemaphoreType.DMA,
            pltpu.SemaphoreType.DMA,
            pltpu.SemaphoreType.DMA,
        ],
    )


_sc_scatter = _make_scatter(HID)
_sc_scatter64 = _make_scatter(OUT)


_BLK = 512
_NBLK = NP // _BLK


def _prep_body(parts_ref, x_ref, cm_ref, hn0_ref, hn1_ref, hn2_ref):
    deg = jnp.sum(parts_ref[...], axis=0)
    c = jnp.where(deg > 0, lax.rsqrt(deg), 0.0)
    c8 = jnp.concatenate(
        [c[1:2], c[3:4], c[5:6], c[0:1], c[2:3], c[4:5],
         jnp.zeros((2, NP), jnp.float32)], axis=0)
    cm = c8.T
    cm_ref[...] = cm
    x = x_ref[...]
    hn0_ref[...] = x * cm[:, 3:4]
    hn1_ref[...] = x * cm[:, 4:5]
    hn2_ref[...] = x * cm[:, 5:6]


_tc_prep = pl.pallas_call(
    _prep_body,
    out_shape=[jax.ShapeDtypeStruct((NP, 8), jnp.float32)]
    + [jax.ShapeDtypeStruct((NP, IN), jnp.float32)] * 3,
)


def _layer_compute(pa_ref, pb_ref, cm_ref, w_ref, b_ref):
    cm = cm_ref[...]
    p = pa_ref[...] + pb_ref[...]
    w = w_ref[...]
    h = (jnp.dot(p[0] * cm[:, 0:1], w[0], preferred_element_type=jnp.float32)
         + jnp.dot(p[1] * cm[:, 1:2], w[1], preferred_element_type=jnp.float32)
         + jnp.dot(p[2] * cm[:, 2:3], w[2], preferred_element_type=jnp.float32)
         + jnp.sum(b_ref[...], axis=0, keepdims=True))
    return h, cm


def _layer_body(pa_ref, pb_ref, cm_ref, w_ref, b_ref,
                hn0_ref, hn1_ref, hn2_ref):
    h, cm = _layer_compute(pa_ref, pb_ref, cm_ref, w_ref, b_ref)
    h = jnp.maximum(h, 0.0)
    hn0_ref[...] = h * cm[:, 3:4]
    hn1_ref[...] = h * cm[:, 4:5]
    hn2_ref[...] = h * cm[:, 5:6]


_tc_layer = pl.pallas_call(
    _layer_body,
    grid=(_NBLK,),
    in_specs=[pl.BlockSpec((3, _BLK, HID), lambda i: (0, i, 0)),
              pl.BlockSpec((3, _BLK, HID), lambda i: (0, i, 0)),
              pl.BlockSpec((_BLK, 8), lambda i: (i, 0)),
              pl.BlockSpec((3, HID, HID), lambda i: (0, 0, 0)),
              pl.BlockSpec((3, HID), lambda i: (0, 0))],
    out_specs=[pl.BlockSpec((_BLK, HID), lambda i: (i, 0))] * 3,
    out_shape=[jax.ShapeDtypeStruct((NP, HID), jnp.float32)] * 3,
)


def _layer2m_body(pa_ref, pb_ref, cm_ref, w_ref, b_ref, w3_ref,
                  m0_ref, m1_ref, m2_ref):
    h, cm = _layer_compute(pa_ref, pb_ref, cm_ref, w_ref, b_ref)
    h = jnp.maximum(h, 0.0)
    w3 = w3_ref[...]
    m0_ref[...] = jnp.dot(h * cm[:, 3:4], w3[0],
                          preferred_element_type=jnp.float32)
    m1_ref[...] = jnp.dot(h * cm[:, 4:5], w3[1],
                          preferred_element_type=jnp.float32)
    m2_ref[...] = jnp.dot(h * cm[:, 5:6], w3[2],
                          preferred_element_type=jnp.float32)


_tc_layer2m = pl.pallas_call(
    _layer2m_body,
    grid=(_NBLK,),
    in_specs=[pl.BlockSpec((3, _BLK, HID), lambda i: (0, i, 0)),
              pl.BlockSpec((3, _BLK, HID), lambda i: (0, i, 0)),
              pl.BlockSpec((_BLK, 8), lambda i: (i, 0)),
              pl.BlockSpec((3, HID, HID), lambda i: (0, 0, 0)),
              pl.BlockSpec((3, HID), lambda i: (0, 0)),
              pl.BlockSpec((3, HID, OUT), lambda i: (0, 0, 0))],
    out_specs=[pl.BlockSpec((_BLK, OUT), lambda i: (i, 0))] * 3,
    out_shape=[jax.ShapeDtypeStruct((NP, OUT), jnp.float32)] * 3,
)


def _fin_body(pa_ref, pb_ref, cm_ref, b_ref, h_ref):
    cm = cm_ref[...]
    p = pa_ref[...] + pb_ref[...]
    h_ref[...] = (p[0] * cm[:, 0:1] + p[1] * cm[:, 1:2] + p[2] * cm[:, 2:3]
                  + jnp.sum(b_ref[...], axis=0, keepdims=True))


_tc_fin = pl.pallas_call(
    _fin_body,
    grid=(_NBLK,),
    in_specs=[pl.BlockSpec((3, _BLK, OUT), lambda i: (0, i, 0)),
              pl.BlockSpec((3, _BLK, OUT), lambda i: (0, i, 0)),
              pl.BlockSpec((_BLK, 8), lambda i: (i, 0)),
              pl.BlockSpec((3, OUT), lambda i: (0, 0))],
    out_specs=pl.BlockSpec((_BLK, OUT), lambda i: (i, 0)),
    out_shape=jax.ShapeDtypeStruct((NP, OUT), jnp.float32),
)


def kernel(x, edge_index_r0, edge_index_r1, edge_index_r2,
           W0, b0, W1, b1, W2, b2, W3, b3):
    edges = []
    edges_flat = []
    pad_ids = (N + jnp.arange(EP - E, dtype=jnp.int32) % (NP - N))
    for ei in (edge_index_r0, edge_index_r1, edge_index_r2):
        for side in range(2):
            idx = jnp.concatenate([ei[side].astype(jnp.int32), pad_ids])
            edges_flat.append(idx)
            edges.append(idx.reshape(NW, IRPW, 128))

    parts = _sc_degrees(*edges_flat)
    xp = jnp.pad(x, ((0, NP - N), (0, 0)))
    cmat, hn0, hn1, hn2 = _tc_prep(parts.reshape(NW, 6, NP), xp)

    for l in range(2):
        pa, pb = _sc_scatter(*edges, hn0, hn1, hn2)
        hn0, hn1, hn2 = _tc_layer(pa, pb, cmat, (W0, W1)[l], (b0, b1)[l])
    pa, pb = _sc_scatter(*edges, hn0, hn1, hn2)
    m0, m1, m2 = _tc_layer2m(pa, pb, cmat, W2, b2, W3)
    pa, pb = _sc_scatter64(*edges, m0, m1, m2)
    h = _tc_fin(pa, pb, cmat, b3)
    return h[:N]


# speedup vs baseline: 1.0470x; 1.0009x over previous
import jax
import jax.numpy as jnp
from jax import lax
from jax.experimental import pallas as pl
from jax.experimental.pallas import tpu as pltpu
from jax.experimental.pallas import tpu_sc as plsc

N = 10000
NP = 10240
E = 100000
EP = 106496
NC, NS = 2, 16
NW = NC * NS
IRPW = 26
RPT = NP // NS

IN, HID, OUT = 128, 128, 64

_MESH = plsc.VectorSubcoreMesh(
    core_axis_name="c", subcore_axis_name="s", num_cores=NC, num_subcores=NS)


EPW = EP // NW


def _deg_body(s0, d0, s1, d1, s2, d2, out, idx0_v, idx1_v, hist_v, dsem):
    cid = lax.axis_index("c")
    sid = lax.axis_index("s")
    w = cid * NS + sid
    zeros16 = jnp.zeros((16,), jnp.float32)
    ones16 = jnp.ones((16,), jnp.float32)

    def zbody(i, _):
        hist_v[pl.ds(i * 16, 16)] = zeros16
        return 0
    lax.fori_loop(0, 6 * NP // 16, zbody, 0)

    arrs = (s0, d0, s1, d1, s2, d2)
    bufs = (idx0_v, idx1_v)
    pltpu.async_copy(arrs[0].at[pl.ds(w * EPW, EPW)], bufs[0], dsem)
    for h, arr in enumerate(arrs):
        buf = bufs[h % 2]
        pltpu.make_async_copy(arr.at[pl.ds(0, EPW)], buf, dsem).wait()
        if h + 1 < 6:
            pltpu.async_copy(arrs[h + 1].at[pl.ds(w * EPW, EPW)],
                             bufs[(h + 1) % 2], dsem)

        def gbody(g, _, h=h, buf=buf):
            vec0 = buf[pl.ds(g * 32, 16)]
            vec1 = buf[pl.ds(g * 32 + 16, 16)]
            plsc.addupdate_scatter(hist_v, [vec0 + jnp.int32(h * NP)], ones16)
            plsc.addupdate_scatter(hist_v, [vec1 + jnp.int32(h * NP)], ones16)
            return 0
        lax.fori_loop(0, EPW // 32, gbody, 0)

    pltpu.sync_copy(hist_v, out.at[pl.ds(w * 6 * NP, 6 * NP)])


_sc_degrees = pl.kernel(
    _deg_body,
    out_type=jax.ShapeDtypeStruct((NW * 6 * NP,), jnp.float32),
    mesh=_MESH,
    compiler_params=pltpu.CompilerParams(needs_layout_passes=False),
    scratch_types=[
        pltpu.VMEM((EPW,), jnp.int32),
        pltpu.VMEM((EPW,), jnp.int32),
        pltpu.VMEM((6 * NP,), jnp.float32),
        pltpu.SemaphoreType.DMA,
    ],
)


def _scat_body(s0, d0, s1, d1, s2, d2, hn0, hn1, hn2,
                 out_a, out_b, idxs_v, idxd_v, rows_v, zeros_v, acc_sh,
                 sem, gsem0, gsem1, ssem0, ssem1):
    cid = lax.axis_index("c")
    sid = lax.axis_index("s")
    w = cid * NS + sid
    zeros16 = jnp.zeros((16,), jnp.float32)
    gsem = (gsem0, gsem1)
    ssem = (ssem0, ssem1)

    def zbody(i, _):
        zeros_v[i // 8, pl.ds((i % 8) * 16, 16)] = zeros16
        return 0
    lax.fori_loop(0, 32 * 8, zbody, 0)

    for r, (se, de, hn) in enumerate(((s0, d0, hn0), (s1, d1, hn1),
                                      (s2, d2, hn2))):
        zds = [pltpu.async_copy(
                   zeros_v, acc_sh.at[pl.ds(sid * RPT + j * 32, 32)], sem)
               for j in range(RPT // 32)]
        pltpu.sync_copy(se.at[w], idxs_v)
        pltpu.sync_copy(de.at[w], idxd_v)

        def fire_g(i, b, hn=hn):
            pltpu.async_copy(hn.at[idxs_v.at[i]],
                             rows_v.at[pl.ds(b * 128, 128)], gsem[b])

        def fire_s(i, b):
            pltpu.async_copy(rows_v.at[pl.ds(b * 128, 128)],
                             acc_sh.at[idxd_v.at[i]], ssem[b], add=True)

        def wait_g(b, hn=hn):
            pltpu.make_async_copy(hn.at[pl.ds(0, 128)],
                                  rows_v.at[pl.ds(b * 128, 128)],
                                  gsem[b]).wait()

        def wait_s(b):
            pltpu.make_async_copy(rows_v.at[pl.ds(b * 128, 128)],
                                  acc_sh.at[pl.ds(0, 128)], ssem[b]).wait()

        fire_g(0, 0)
        fire_g(1, 1)
        for d in zds:
            d.wait()
        plsc.subcore_barrier()
        wait_g(0)
        fire_s(0, 0)

        def ebody(t, _):
            g1 = 2 * t + 1
            wait_g(1)
            wait_s(0)
            fire_g(g1 + 1, 0)
            fire_s(g1, 1)
            wait_g(0)
            wait_s(1)
            fire_g(g1 + 2, 1)
            fire_s(g1 + 1, 0)
            return 0
        lax.fori_loop(0, (IRPW - 2) // 2, ebody, 0)

        wait_g(1)
        wait_s(0)
        fire_s(IRPW - 1, 1)
        wait_s(1)
        plsc.subcore_barrier()

        @pl.when(cid == 0)
        def _():
            pltpu.sync_copy(acc_sh.at[pl.ds(sid * RPT, RPT)],
                            out_a.at[r, pl.ds(sid * RPT, RPT)])

        @pl.when(cid == 1)
        def _():
            pltpu.sync_copy(acc_sh.at[pl.ds(sid * RPT, RPT)],
                            out_b.at[r, pl.ds(sid * RPT, RPT)])
        plsc.subcore_barrier()


def _make_scatter(width):
    return pl.kernel(
        _scat_body,
        out_type=[jax.ShapeDtypeStruct((3, NP, width), jnp.float32),
                  jax.ShapeDtypeStruct((3, NP, width), jnp.float32)],
        mesh=_MESH,
        compiler_params=pltpu.CompilerParams(
            use_tc_tiling_on_sc=(width == 128)),
        scratch_types=[
            pltpu.VMEM((IRPW, 128), jnp.int32),
            pltpu.VMEM((IRPW, 128), jnp.int32),
            pltpu.VMEM((256, width), jnp.float32),
            pltpu.VMEM((32, width), jnp.float32),
            pltpu.VMEM_SHARED((NP, width), jnp.float32),
            pltpu.SemaphoreType.DMA,
            pltpu.SemaphoreType.DMA,
            pltpu.SemaphoreType.DMA,
            pltpu.SemaphoreType.DMA,
            pltpu.SemaphoreType.DMA,
        ],
    )


_sc_scatter = _make_scatter(HID)
_sc_scatter64 = _make_scatter(OUT)


_BLK = 512
_NBLK = NP // _BLK


def _prep_body(parts_ref, x_ref, cm_ref, hn0_ref, hn1_ref, hn2_ref):
    deg = jnp.sum(parts_ref[...], axis=0)
    c = jnp.where(deg > 0, lax.rsqrt(deg), 0.0)
    c8 = jnp.concatenate(
        [c[1:2], c[3:4], c[5:6], c[0:1], c[2:3], c[4:5],
         jnp.zeros((2, NP), jnp.float32)], axis=0)
    cm = c8.T
    cm_ref[...] = cm
    x = x_ref[...]
    hn0_ref[...] = x * cm[:, 3:4]
    hn1_ref[...] = x * cm[:, 4:5]
    hn2_ref[...] = x * cm[:, 5:6]


_tc_prep = pl.pallas_call(
    _prep_body,
    out_shape=[jax.ShapeDtypeStruct((NP, 8), jnp.float32)]
    + [jax.ShapeDtypeStruct((NP, IN), jnp.float32)] * 3,
)


def _layer_compute(pa_ref, pb_ref, cm_ref, w_ref, b_ref):
    cm = cm_ref[...]
    p = pa_ref[...] + pb_ref[...]
    w = w_ref[...]
    h = (jnp.dot(p[0] * cm[:, 0:1], w[0], preferred_element_type=jnp.float32)
         + jnp.dot(p[1] * cm[:, 1:2], w[1], preferred_element_type=jnp.float32)
         + jnp.dot(p[2] * cm[:, 2:3], w[2], preferred_element_type=jnp.float32)
         + jnp.sum(b_ref[...], axis=0, keepdims=True))
    return h, cm


def _layer_body(pa_ref, pb_ref, cm_ref, w_ref, b_ref,
                hn0_ref, hn1_ref, hn2_ref):
    h, cm = _layer_compute(pa_ref, pb_ref, cm_ref, w_ref, b_ref)
    h = jnp.maximum(h, 0.0)
    hn0_ref[...] = h * cm[:, 3:4]
    hn1_ref[...] = h * cm[:, 4:5]
    hn2_ref[...] = h * cm[:, 5:6]


_tc_layer = pl.pallas_call(
    _layer_body,
    grid=(_NBLK,),
    in_specs=[pl.BlockSpec((3, _BLK, HID), lambda i: (0, i, 0)),
              pl.BlockSpec((3, _BLK, HID), lambda i: (0, i, 0)),
              pl.BlockSpec((_BLK, 8), lambda i: (i, 0)),
              pl.BlockSpec((3, HID, HID), lambda i: (0, 0, 0)),
              pl.BlockSpec((3, HID), lambda i: (0, 0))],
    out_specs=[pl.BlockSpec((_BLK, HID), lambda i: (i, 0))] * 3,
    out_shape=[jax.ShapeDtypeStruct((NP, HID), jnp.float32)] * 3,
)


def _layer2m_body(pa_ref, pb_ref, cm_ref, w_ref, b_ref, w3_ref,
                  m0_ref, m1_ref, m2_ref):
    h, cm = _layer_compute(pa_ref, pb_ref, cm_ref, w_ref, b_ref)
    h = jnp.maximum(h, 0.0)
    w3 = w3_ref[...]
    m0_ref[...] = jnp.dot(h * cm[:, 3:4], w3[0],
                          preferred_element_type=jnp.float32)
    m1_ref[...] = jnp.dot(h * cm[:, 4:5], w3[1],
                          preferred_element_type=jnp.float32)
    m2_ref[...] = jnp.dot(h * cm[:, 5:6], w3[2],
                          preferred_element_type=jnp.float32)


_tc_layer2m = pl.pallas_call(
    _layer2m_body,
    grid=(_NBLK,),
    in_specs=[pl.BlockSpec((3, _BLK, HID), lambda i: (0, i, 0)),
              pl.BlockSpec((3, _BLK, HID), lambda i: (0, i, 0)),
              pl.BlockSpec((_BLK, 8), lambda i: (i, 0)),
              pl.BlockSpec((3, HID, HID), lambda i: (0, 0, 0)),
              pl.BlockSpec((3, HID), lambda i: (0, 0)),
              pl.BlockSpec((3, HID, OUT), lambda i: (0, 0, 0))],
    out_specs=[pl.BlockSpec((_BLK, OUT), lambda i: (i, 0))] * 3,
    out_shape=[jax.ShapeDtypeStruct((NP, OUT), jnp.float32)] * 3,
)


def _fin_body(pa_ref, pb_ref, cm_ref, b_ref, h_ref):
    cm = cm_ref[...]
    p = pa_ref[...] + pb_ref[...]
    h_ref[...] = (p[0] * cm[:, 0:1] + p[1] * cm[:, 1:2] + p[2] * cm[:, 2:3]
                  + jnp.sum(b_ref[...], axis=0, keepdims=True))


_tc_fin = pl.pallas_call(
    _fin_body,
    grid=(_NBLK,),
    in_specs=[pl.BlockSpec((3, _BLK, OUT), lambda i: (0, i, 0)),
              pl.BlockSpec((3, _BLK, OUT), lambda i: (0, i, 0)),
              pl.BlockSpec((_BLK, 8), lambda i: (i, 0)),
              pl.BlockSpec((3, OUT), lambda i: (0, 0))],
    out_specs=pl.BlockSpec((_BLK, OUT), lambda i: (i, 0)),
    out_shape=jax.ShapeDtypeStruct((NP, OUT), jnp.float32),
)


def kernel(x, edge_index_r0, edge_index_r1, edge_index_r2,
           W0, b0, W1, b1, W2, b2, W3, b3):
    edges = []
    edges_flat = []
    pad_ids = (N + jnp.arange(EP - E, dtype=jnp.int32) % (NP - N))
    for ei in (edge_index_r0, edge_index_r1, edge_index_r2):
        for side in range(2):
            idx = jnp.concatenate([ei[side].astype(jnp.int32), pad_ids])
            edges_flat.append(idx)
            edges.append(idx.reshape(NW, IRPW, 128))

    parts = _sc_degrees(*edges_flat)
    xp = jnp.pad(x, ((0, NP - N), (0, 0)))
    cmat, hn0, hn1, hn2 = _tc_prep(parts.reshape(NW, 6, NP), xp)

    for l in range(2):
        pa, pb = _sc_scatter(*edges, hn0, hn1, hn2)
        hn0, hn1, hn2 = _tc_layer(pa, pb, cmat, (W0, W1)[l], (b0, b1)[l])
    pa, pb = _sc_scatter(*edges, hn0, hn1, hn2)
    m0, m1, m2 = _tc_layer2m(pa, pb, cmat, W2, b2, W3)
    pa, pb = _sc_scatter64(*edges, m0, m1, m2)
    h = _tc_fin(pa, pb, cmat, b3)
    return h[:N]
